# baseline (speedup 1.0000x reference)
import functools

import jax
import jax.numpy as jnp
from jax import lax
from jax.experimental import pallas as pl
from jax.experimental.pallas import tpu as pltpu
from jax.experimental.pallas import tpu_sc as plsc

H, W = 1024, 2048
CROP_H, CROP_W = 720, 1280
SAMPLES = 4
LABEL_COUNT = 19
NC, NS = 2, 16
NWORK = NC * NS
RPW = 23
WB = 1296
NVEC = WB // 16
NBANK = 4
HIST_W = LABEL_COUNT * 16 * NBANK

_mesh = plsc.VectorSubcoreMesh(core_axis_name="c", subcore_axis_name="s")
_params = pltpu.CompilerParams(
    use_tc_tiling_on_sc=False, needs_layout_passes=False)


_TOPS = (219, 196, 73, 29)
_LEFTS = (192, 367, 42, 696)


def _pick(vec, iota, k):
    return jnp.max(jnp.where(iota == k, vec, 0))


@functools.partial(
    pl.kernel,
    out_type=jax.ShapeDtypeStruct((NWORK, SAMPLES * HIST_W), jnp.int32),
    mesh=_mesh,
    scratch_types=[
        pltpu.VMEM((RPW, WB), jnp.int32),
        pltpu.VMEM((SAMPLES * HIST_W,), jnp.int32),
    ],
    compiler_params=_params,
)
def _hist_kernel(label_hbm, out_hbm, buf_v, hist_v):
    w = lax.axis_index("s") * NC + lax.axis_index("c")
    iota = lax.iota(jnp.int32, 16)
    zeros = jnp.zeros((16,), jnp.int32)
    ones = jnp.ones((16,), jnp.int32)

    for k in range(SAMPLES * HIST_W // 16):
        hist_v[pl.ds(k * 16, 16)] = zeros

    lo = jnp.minimum(RPW * w, CROP_H - RPW)
    r_begin = RPW * w

    for c in range(SAMPLES):
        top, left = _TOPS[c], _LEFTS[c]
        left_al = min(left & -16, W - WB)
        shift = left - left_al
        mask_first = iota >= shift
        mask_last = iota < shift

        pltpu.sync_copy(
            label_hbm.at[pl.ds(top + lo, RPW), left_al:left_al + WB], buf_v
        )

        def body(i, carry, _c=c, _mf=mask_first, _ml=mask_last):
            rg = lo + i
            valid = jnp.logical_and(rg >= r_begin, rg < CROP_H)
            rmask = jnp.full((16,), valid)
            m_first = jnp.logical_and(rmask, _mf)
            m_last = jnp.logical_and(rmask, _ml)
            for g in range(0, NVEC, 8):
                js = range(g, min(g + 8, NVEC))
                idxs = []
                for j in js:
                    lv = buf_v[i, pl.ds(j * 16, 16)]
                    base = _c * HIST_W + (j % NBANK) * 16 + iota
                    idxs.append(lv * (16 * NBANK) + base)
                for k, j in enumerate(js):
                    m = m_first if j == 0 else (
                        m_last if j == NVEC - 1 else rmask)
                    plsc.addupdate_scatter(hist_v, [idxs[k]], ones, mask=m)
            return carry

        lax.fori_loop(0, RPW, body, 0)

    pltpu.sync_copy(hist_v, out_hbm.at[w])


_N_BLK = CROP_H // 8 + 1


def _tc_crop_body(sel_ref, img_ref, lab_ref, oimg_ref, olab_ref,
                  pimg_ref, plab_ref):
    i = pl.program_id(0)
    rs = sel_ref[1]
    left = sel_ref[2]
    cur_i = img_ref[...]
    cur_l = lab_ref[...]

    @pl.when(i > 0)
    def _():
        wimg = jnp.concatenate([pimg_ref[...], cur_i], axis=1)
        wlab = jnp.concatenate([plab_ref[...], cur_l], axis=0)
        wimg = pltpu.roll(wimg, 16 - rs, 1)[:, :8, :]
        wlab = pltpu.roll(wlab, 16 - rs, 0)[:8, :]
        oimg_ref[...] = pltpu.roll(wimg, W - left, 2)[:, :, :CROP_W]
        olab_ref[...] = pltpu.roll(wlab, W - left, 1)[:, :CROP_W]

    pimg_ref[...] = cur_i
    plab_ref[...] = cur_l


_tc_crop = pl.pallas_call(
    _tc_crop_body,
    grid_spec=pltpu.PrefetchScalarGridSpec(
        num_scalar_prefetch=1,
        grid=(_N_BLK,),
        in_specs=[
            pl.BlockSpec(
                (3, 8, W), lambda i, sel: (0, jnp.minimum(sel[0] + i, H // 8 - 1), 0)),
            pl.BlockSpec(
                (8, W), lambda i, sel: (jnp.minimum(sel[0] + i, H // 8 - 1), 0)),
        ],
        out_specs=[
            pl.BlockSpec(
                (3, 8, CROP_W), lambda i, sel: (0, jnp.maximum(i - 1, 0), 0)),
            pl.BlockSpec(
                (8, CROP_W), lambda i, sel: (jnp.maximum(i - 1, 0), 0)),
        ],
        scratch_shapes=[
            pltpu.VMEM((3, 8, W), jnp.float32),
            pltpu.VMEM((8, W), jnp.int32),
        ],
    ),
    out_shape=(
        jax.ShapeDtypeStruct((3, CROP_H, CROP_W), jnp.float32),
        jax.ShapeDtypeStruct((CROP_H, CROP_W), jnp.int32),
    ),
    compiler_params=pltpu.CompilerParams(
        dimension_semantics=("arbitrary",)),
)


def kernel(image, label_image, label_costs):
    label2d = label_image.reshape(H, W)

    parts = _hist_kernel(label2d)
    counts = parts.reshape(
        NWORK, SAMPLES, LABEL_COUNT, NBANK * 16).sum(axis=(0, 3))

    norm_costs = label_costs / jnp.maximum(jnp.sum(jnp.abs(label_costs)), 1e-12)
    total = float(CROP_H * CROP_W)

    def cost_of(c):
        dist = counts[c].astype(jnp.float32) / total
        return jnp.sum(norm_costs * dist)

    best_cost = cost_of(0)
    best_idx = jnp.int32(0)
    for c in range(1, SAMPLES):
        cc = cost_of(c)
        better = cc < best_cost
        best_idx = jnp.where(better, jnp.int32(c), best_idx)
        best_cost = jnp.where(better, cc, best_cost)

    tops_a = jnp.asarray(_TOPS, jnp.int32)
    lefts_a = jnp.asarray(_LEFTS, jnp.int32)
    top = tops_a[best_idx]
    left = lefts_a[best_idx]
    sel = jnp.stack([top // 8, top % 8, left])
    best_image, best_label = _tc_crop(sel, image, label2d)
    return best_image, best_label.reshape(1, CROP_H, CROP_W), best_cost
